# Initial kernel scaffold; baseline (speedup 1.0000x reference)
import jax
import jax.numpy as jnp
from jax.experimental import pallas as pl


def kernel(hidden_states, router_w, gate_w, up_w, down_w):
    raise NotImplementedError("write your pallas kernel here")


# speedup vs baseline: 1.6981x; 1.6981x over previous
import functools

import jax
import jax.numpy as jnp
from jax.experimental import pallas as pl
from jax.experimental.pallas import tpu as pltpu

HID = 1024
FFD = 2816
NE = 8
NT = 2048
BT = 256
NB = (2 * NT + NE * BT) // BT
PAD = NB * BT
NF = 2
BF = FFD // NF


def _router_body(x_ref, rw_ref, logits_ref, probs_ref, i0_ref, i1_ref,
                 w0_ref, w1_ref, counts_ref, avg_ref):
    x = x_ref[...]
    rw = rw_ref[...]
    logits = jax.lax.dot_general(x, rw, (((1,), (1,)), ((), ())),
                                 preferred_element_type=jnp.float32)
    logits_ref[...] = logits
    m = jnp.max(logits, axis=1, keepdims=True)
    ex = jnp.exp(logits - m)
    probs = ex / jnp.sum(ex, axis=1, keepdims=True)
    probs_ref[...] = probs
    iota = jax.lax.broadcasted_iota(jnp.int32, probs.shape, 1)
    p1 = jnp.max(probs, axis=1, keepdims=True)
    i0 = jnp.min(jnp.where(probs == p1, iota, NE), axis=1, keepdims=True)
    probs2 = jnp.where(iota == i0, -1.0, probs)
    p2 = jnp.max(probs2, axis=1, keepdims=True)
    i1 = jnp.min(jnp.where(probs2 == p2, iota, NE), axis=1, keepdims=True)
    s = p1 + p2
    w0_ref[...] = p1 / s
    w1_ref[...] = p2 / s
    i0_ref[...] = i0
    i1_ref[...] = i1
    oh = (iota == i0).astype(jnp.float32) + (iota == i1).astype(jnp.float32)
    counts_ref[...] = jnp.sum(oh, axis=0, keepdims=True)
    avg_ref[...] = jnp.mean(probs, axis=0, keepdims=True)


def _router(flat, router_w):
    out = pl.pallas_call(
        _router_body,
        out_shape=(
            jax.ShapeDtypeStruct((NT, NE), jnp.float32),
            jax.ShapeDtypeStruct((NT, NE), jnp.float32),
            jax.ShapeDtypeStruct((NT, 1), jnp.int32),
            jax.ShapeDtypeStruct((NT, 1), jnp.int32),
            jax.ShapeDtypeStruct((NT, 1), jnp.float32),
            jax.ShapeDtypeStruct((NT, 1), jnp.float32),
            jax.ShapeDtypeStruct((1, NE), jnp.float32),
            jax.ShapeDtypeStruct((1, NE), jnp.float32),
        ),
    )(flat, router_w)
    return out


def _ffn_body(be_ref, x_ref, gw_ref, uw_ref, dw_ref, y_ref):
    f = pl.program_id(1)
    x = x_ref[...]
    g = jax.lax.dot_general(x, gw_ref[0], (((1,), (1,)), ((), ())),
                            preferred_element_type=jnp.float32)
    u = jax.lax.dot_general(x, uw_ref[0], (((1,), (1,)), ((), ())),
                            preferred_element_type=jnp.float32)
    h = (g / (1.0 + jnp.exp(-g))) * u
    y = jax.lax.dot_general(h, dw_ref[0], (((1,), (1,)), ((), ())),
                            preferred_element_type=jnp.float32)

    @pl.when(f == 0)
    def _():
        y_ref[...] = y

    @pl.when(f > 0)
    def _():
        y_ref[...] = y_ref[...] + y


def _grouped_ffn(xg, gate_w, up_w, down_w, block_expert):
    grid_spec = pltpu.PrefetchScalarGridSpec(
        num_scalar_prefetch=1,
        grid=(NB, NF),
        in_specs=[
            pl.BlockSpec((BT, HID), lambda i, f, be: (i, 0)),
            pl.BlockSpec((1, BF, HID), lambda i, f, be: (be[i], f, 0)),
            pl.BlockSpec((1, BF, HID), lambda i, f, be: (be[i], f, 0)),
            pl.BlockSpec((1, HID, BF), lambda i, f, be: (be[i], 0, f)),
        ],
        out_specs=pl.BlockSpec((BT, HID), lambda i, f, be: (i, 0)),
    )
    return pl.pallas_call(
        _ffn_body,
        grid_spec=grid_spec,
        out_shape=jax.ShapeDtypeStruct((PAD, HID), jnp.float32),
    )(block_expert, xg, gate_w, up_w, down_w)


def kernel(hidden_states, router_w, gate_w, up_w, down_w):
    b, s, d = hidden_states.shape
    flat = hidden_states.reshape(-1, d)

    logits, probs, i0, i1, w0, w1, counts, avg_prob = _router(flat, router_w)
    i0 = i0[:, 0]
    i1 = i1[:, 0]

    counts_i = counts[0].astype(jnp.int32)
    padded = ((counts_i + BT - 1) // BT) * BT
    ends = jnp.cumsum(padded)
    starts = ends - padded
    e_all = jnp.concatenate([i0, i1])
    oh = jax.nn.one_hot(e_all, NE, dtype=jnp.int32)
    ranks = jnp.cumsum(oh, axis=0) - oh
    rank = jnp.take_along_axis(ranks, e_all[:, None], axis=1)[:, 0]
    dest_all = starts[e_all] + rank
    tok = jnp.arange(NT, dtype=jnp.int32)
    token_src = jnp.zeros((PAD,), jnp.int32).at[dest_all].set(
        jnp.concatenate([tok, tok]))
    block_start = jnp.arange(NB, dtype=jnp.int32) * BT
    block_expert = jnp.minimum(
        jnp.searchsorted(ends, block_start, side='right').astype(jnp.int32),
        NE - 1)

    xg = flat[token_src]
    y = _grouped_ffn(xg, gate_w, up_w, down_w, block_expert)
    d0 = dest_all[:NT]
    d1 = dest_all[NT:]
    out = w0 * y[d0] + w1 * y[d1]

    expert_frac = counts[0] / (NT * 2)
    return (out.reshape(b, s, d), expert_frac, avg_prob[0], logits, probs)


# speedup vs baseline: 1.8360x; 1.0812x over previous
import functools

import jax
import jax.numpy as jnp
from jax.experimental import pallas as pl
from jax.experimental.pallas import tpu as pltpu

HID = 1024
FFD = 2816
NE = 8
NT = 2048
BT = 256
NB = (2 * NT + NE * BT) // BT
PAD = NB * BT
NF = 2
BF = FFD // NF


def _router_body(x_ref, rw_ref, logits_ref, probs_ref, i0_ref, i1_ref,
                 w0_ref, w1_ref, counts_ref, avg_ref):
    x = x_ref[...]
    rw = rw_ref[...]
    logits = jax.lax.dot_general(x, rw, (((1,), (1,)), ((), ())),
                                 preferred_element_type=jnp.float32)
    logits_ref[...] = logits
    m = jnp.max(logits, axis=1, keepdims=True)
    ex = jnp.exp(logits - m)
    probs = ex / jnp.sum(ex, axis=1, keepdims=True)
    probs_ref[...] = probs
    iota = jax.lax.broadcasted_iota(jnp.int32, probs.shape, 1)
    p1 = jnp.max(probs, axis=1, keepdims=True)
    i0 = jnp.min(jnp.where(probs == p1, iota, NE), axis=1, keepdims=True)
    probs2 = jnp.where(iota == i0, -1.0, probs)
    p2 = jnp.max(probs2, axis=1, keepdims=True)
    i1 = jnp.min(jnp.where(probs2 == p2, iota, NE), axis=1, keepdims=True)
    s = p1 + p2
    w0_ref[...] = p1 / s
    w1_ref[...] = p2 / s
    i0_ref[...] = i0
    i1_ref[...] = i1
    oh = (iota == i0).astype(jnp.float32) + (iota == i1).astype(jnp.float32)
    counts_ref[...] = jnp.sum(oh, axis=0, keepdims=True)
    avg_ref[...] = jnp.mean(probs, axis=0, keepdims=True)


def _router(flat, router_w):
    out = pl.pallas_call(
        _router_body,
        out_shape=(
            jax.ShapeDtypeStruct((NT, NE), jnp.float32),
            jax.ShapeDtypeStruct((NT, NE), jnp.float32),
            jax.ShapeDtypeStruct((NT, 1), jnp.int32),
            jax.ShapeDtypeStruct((NT, 1), jnp.int32),
            jax.ShapeDtypeStruct((NT, 1), jnp.float32),
            jax.ShapeDtypeStruct((NT, 1), jnp.float32),
            jax.ShapeDtypeStruct((1, NE), jnp.float32),
            jax.ShapeDtypeStruct((1, NE), jnp.float32),
        ),
    )(flat, router_w)
    return out


def _ffn_body(be_ref, x_ref, gw_ref, uw_ref, dw_ref, y_ref):
    x = x_ref[...]
    g = jax.lax.dot_general(x, gw_ref[0], (((1,), (1,)), ((), ())),
                            preferred_element_type=jnp.float32)
    u = jax.lax.dot_general(x, uw_ref[0], (((1,), (1,)), ((), ())),
                            preferred_element_type=jnp.float32)
    h = (g / (1.0 + jnp.exp(-g))) * u
    y_ref[0] = jax.lax.dot_general(h, dw_ref[0], (((1,), (1,)), ((), ())),
                                   preferred_element_type=jnp.float32)


def _grouped_ffn(xg, gate_w, up_w, down_w, block_expert):
    grid_spec = pltpu.PrefetchScalarGridSpec(
        num_scalar_prefetch=1,
        grid=(NF, NB),
        in_specs=[
            pl.BlockSpec((BT, HID), lambda f, i, be: (i, 0)),
            pl.BlockSpec((1, BF, HID), lambda f, i, be: (be[i], f, 0)),
            pl.BlockSpec((1, BF, HID), lambda f, i, be: (be[i], f, 0)),
            pl.BlockSpec((1, HID, BF), lambda f, i, be: (be[i], 0, f)),
        ],
        out_specs=pl.BlockSpec((1, BT, HID), lambda f, i, be: (f, i, 0)),
    )
    return pl.pallas_call(
        _ffn_body,
        grid_spec=grid_spec,
        out_shape=jax.ShapeDtypeStruct((NF, PAD, HID), jnp.float32),
    )(block_expert, xg, gate_w, up_w, down_w)


def kernel(hidden_states, router_w, gate_w, up_w, down_w):
    b, s, d = hidden_states.shape
    flat = hidden_states.reshape(-1, d)

    logits, probs, i0, i1, w0, w1, counts, avg_prob = _router(flat, router_w)
    i0 = i0[:, 0]
    i1 = i1[:, 0]

    counts_i = counts[0].astype(jnp.int32)
    padded = ((counts_i + BT - 1) // BT) * BT
    ends = jnp.cumsum(padded)
    starts = ends - padded
    e_all = jnp.concatenate([i0, i1])
    oh = jax.nn.one_hot(e_all, NE, dtype=jnp.int32)
    ranks = jnp.cumsum(oh, axis=0) - oh
    rank = jnp.take_along_axis(ranks, e_all[:, None], axis=1)[:, 0]
    dest_all = starts[e_all] + rank
    tok = jnp.arange(NT, dtype=jnp.int32)
    token_src = jnp.zeros((PAD,), jnp.int32).at[dest_all].set(
        jnp.concatenate([tok, tok]))
    block_start = jnp.arange(NB, dtype=jnp.int32) * BT
    block_expert = jnp.minimum(
        jnp.searchsorted(ends, block_start, side='right').astype(jnp.int32),
        NE - 1)

    xg = flat[token_src]
    y = _grouped_ffn(xg, gate_w, up_w, down_w, block_expert)
    d0 = dest_all[:NT]
    d1 = dest_all[NT:]
    ysum = y[0] + y[1]
    out = w0 * ysum[d0] + w1 * ysum[d1]

    expert_frac = counts[0] / (NT * 2)
    return (out.reshape(b, s, d), expert_frac, avg_prob[0], logits, probs)
